# Initial kernel scaffold; baseline (speedup 1.0000x reference)
import jax
import jax.numpy as jnp
from jax.experimental import pallas as pl


def kernel(lidar_out, lidar_pos, lidar_x, lidar_spatial_edge_index, radar1_out, radar1_pos, radar1_x, radar1_batch, radar1_temporal_edge_index, radar1_to_lidar_src, radar1_to_lidar_dst, radar2_out, radar2_pos, radar2_x, radar2_batch, radar2_temporal_edge_index, radar2_to_lidar_src, radar2_to_lidar_dst, dt_sec, gt_radar_pos):
    raise NotImplementedError("write your pallas kernel here")


# speedup vs baseline: 13.7172x; 13.7172x over previous
import functools
import math

import jax
import jax.numpy as jnp
from jax import lax
from jax.experimental import pallas as pl
from jax.experimental.pallas import tpu as pltpu
from jax.experimental.pallas import tpu_sc as plsc

_SCALE_POSE = 10.0
_SCALE_RADAR_V = 5.0
_L_MIN = 2 * math.log(0.03 / _SCALE_POSE + 1e-09)
_L_MAX = 2 * math.log(0.5 / _SCALE_POSE + 1e-09)
_R_MIN = 2 * math.log(0.1 / _SCALE_RADAR_V + 1e-09)
_R_MAX = 2 * math.log(5.0 / _SCALE_RADAR_V + 1e-09)
_GHOST = (0.6 / _SCALE_POSE) ** 2

_NL = 100000
_EL = 1600000
_NR = 20000
_NRP = 20480
_ETP = 128000
_ECP = 256000
_NGT = 256
_D = 8
_CH = 2000

_mesh = plsc.VectorSubcoreMesh(core_axis_name="c", subcore_axis_name="s")
_sc_params = pltpu.CompilerParams(use_tc_tiling_on_sc=False)
_sc_params_nl = pltpu.CompilerParams(
    use_tc_tiling_on_sc=False, needs_layout_passes=False)


@functools.partial(
    pl.kernel,
    out_type=(
        jax.ShapeDtypeStruct((2, _NL, _D), jnp.float32),
        jax.ShapeDtypeStruct((2, _NRP, _D), jnp.float32),
        jax.ShapeDtypeStruct((2, _NRP, _D), jnp.float32),
        jax.ShapeDtypeStruct((_ETP, _D), jnp.float32),
        jax.ShapeDtypeStruct((_ETP, _D), jnp.float32),
        jax.ShapeDtypeStruct((_ETP, _D), jnp.float32),
        jax.ShapeDtypeStruct((_ETP, _D), jnp.float32),
    ),
    scratch_types=[
        pltpu.VMEM((_CH,), jnp.int32),
        pltpu.VMEM((_CH,), jnp.int32),
        pltpu.VMEM((_CH, _D), jnp.float32),
        pltpu.VMEM_SHARED((_NL, _D), jnp.float32),
        pltpu.VMEM_SHARED((_NRP, _D), jnp.float32),
        pltpu.VMEM_SHARED((_NRP, _D), jnp.float32),
        pltpu.SemaphoreType.DMA,
    ],
    mesh=_mesh,
    compiler_params=_sc_params,
)
def _sc_edge_pass(t_lid, t_r1, t_r2, lid_ei, r1cs, r1cd, r2cs, r2cd, tei1,
                  tei2, zeros, lid_acc, r1_acc, r2_acc, g1s, g1d, g2s, g2d,
                  src_v, dst_v, rows_v, accl, acc1, acc2, sem):
    c = lax.axis_index("c")
    s = lax.axis_index("s")
    wid = c * 16 + s

    nl16 = _NL // 16
    nr16 = _NRP // 16
    pltpu.sync_copy(zeros.at[pl.ds(s * nl16, nl16)], accl.at[pl.ds(s * nl16, nl16)])
    pltpu.sync_copy(zeros.at[pl.ds(s * nr16, nr16)], acc1.at[pl.ds(s * nr16, nr16)])
    pltpu.sync_copy(zeros.at[pl.ds(s * nr16, nr16)], acc2.at[pl.ds(s * nr16, nr16)])
    plsc.subcore_barrier()

    ept_l = _EL // 32
    def lid_chunk(k, _):
        base = wid * ept_l + k * _CH
        pltpu.sync_copy(lid_ei.at[0, pl.ds(base, _CH)], src_v)
        pltpu.sync_copy(lid_ei.at[1, pl.ds(base, _CH)], dst_v)
        pltpu.async_copy(t_lid.at[src_v], rows_v, sem).wait()
        pltpu.sync_copy(rows_v, accl.at[dst_v], add=True)
        return 0
    lax.fori_loop(0, ept_l // _CH, lid_chunk, 0)

    ept_c = _ECP // 32
    for cs_ref, cd_ref, acc in ((r1cs, r1cd, acc1), (r2cs, r2cd, acc2)):
        def cross_chunk(k, _, cs_ref=cs_ref, cd_ref=cd_ref, acc=acc):
            base = wid * ept_c + k * _CH
            pltpu.sync_copy(cd_ref.at[pl.ds(base, _CH)], src_v)
            pltpu.sync_copy(cs_ref.at[pl.ds(base, _CH)], dst_v)
            pltpu.async_copy(t_lid.at[src_v], rows_v, sem).wait()
            pltpu.sync_copy(rows_v, acc.at[dst_v], add=True)
            return 0
        lax.fori_loop(0, ept_c // _CH, cross_chunk, 0)

    ept_t = _ETP // 32
    for tei, t_r, gs, gd in ((tei1, t_r1, g1s, g1d), (tei2, t_r2, g2s, g2d)):
        def temp_chunk(k, _, tei=tei, t_r=t_r, gs=gs, gd=gd):
            base = wid * ept_t + k * _CH
            pltpu.sync_copy(tei.at[0, pl.ds(base, _CH)], src_v)
            pltpu.async_copy(t_r.at[src_v], rows_v, sem).wait()
            pltpu.sync_copy(rows_v, gs.at[pl.ds(base, _CH)])
            pltpu.sync_copy(tei.at[1, pl.ds(base, _CH)], dst_v)
            pltpu.async_copy(t_r.at[dst_v], rows_v, sem).wait()
            pltpu.sync_copy(rows_v, gd.at[pl.ds(base, _CH)])
            return 0
        lax.fori_loop(0, ept_t // _CH, temp_chunk, 0)

    plsc.subcore_barrier()
    pltpu.sync_copy(accl.at[pl.ds(s * nl16, nl16)], lid_acc.at[c, pl.ds(s * nl16, nl16)])
    pltpu.sync_copy(acc1.at[pl.ds(s * nr16, nr16)], r1_acc.at[c, pl.ds(s * nr16, nr16)])
    pltpu.sync_copy(acc2.at[pl.ds(s * nr16, nr16)], r2_acc.at[c, pl.ds(s * nr16, nr16)])


@functools.partial(
    pl.kernel,
    out_type=jax.ShapeDtypeStruct((2, 16), jnp.float32),
    scratch_types=[
        pltpu.VMEM((_CH,), jnp.int32),
        pltpu.VMEM((_CH,), jnp.float32),
        pltpu.VMEM((_NR,), jnp.int32),
        pltpu.VMEM((_NR,), jnp.float32),
        pltpu.VMEM((16,), jnp.float32),
    ],
    mesh=_mesh,
    compiler_params=_sc_params_nl,
)
def _sc_phys_pass(tei1, md1, den1, tei2, md2, den2, out,
                  src_v, md_v, eid_v, den_v, ovec):
    c = lax.axis_index("c")
    s = lax.axis_index("s")
    ne = 100000

    def run_branch(tei, md, den, row):
        zi = jnp.zeros((16,), jnp.int32)
        def init_body(i, _):
            eid_v[pl.ds(i * 16, 16)] = zi
            return 0
        lax.fori_loop(0, _NR // 16, init_body, 0)
        pltpu.sync_copy(den, den_v)
        lanes = lax.iota(jnp.int32, 16)

        def p1_chunk(k, _):
            pltpu.sync_copy(tei.at[0, pl.ds(k * _CH, _CH)], src_v)
            def p1_vreg(j, _):
                idx = src_v[pl.ds(j * 16, 16)]
                my = (k * _CH + j * 16 + 1) + lanes
                plsc.store_scatter(eid_v, [idx], my)
                def rnd(r, _):
                    g = plsc.load_gather(eid_v, [idx])
                    m = my > g
                    @pl.when(jnp.any(m))
                    def _():
                        plsc.store_scatter(eid_v, [idx], my, mask=m)
                    return 0
                lax.fori_loop(0, 3, rnd, 0)
                return 0
            lax.fori_loop(0, _CH // 16, p1_vreg, 0)
            return 0
        lax.fori_loop(0, ne // _CH, p1_chunk, 0)

        def p2_chunk(k, acc):
            pltpu.sync_copy(tei.at[0, pl.ds(k * _CH, _CH)], src_v)
            pltpu.sync_copy(md.at[pl.ds(k * _CH, _CH)], md_v)
            def p2_vreg(j, acc):
                idx = src_v[pl.ds(j * 16, 16)]
                my = (k * _CH + j * 16 + 1) + lanes
                g = plsc.load_gather(eid_v, [idx])
                w = g == my
                dg = plsc.load_gather(den_v, [idx])
                mdv = md_v[pl.ds(j * 16, 16)]
                return acc + jnp.where(w, mdv / dg, 0.0)
            return lax.fori_loop(0, _CH // 16, p2_vreg, acc)
        acc = lax.fori_loop(0, ne // _CH, p2_chunk, jnp.zeros((16,), jnp.float32))
        ovec[...] = acc
        pltpu.sync_copy(ovec, out.at[row])

    @pl.when(jnp.logical_and(c == 0, s == 0))
    def _():
        run_branch(tei1, md1, den1, 0)

    @pl.when(jnp.logical_and(c == 1, s == 0))
    def _():
        run_branch(tei2, md2, den2, 1)


def _k0a_body(pos_ref, x_ref, out_ref):
    pos = pos_ref[...]
    x2 = x_ref[:, 2:3]
    sq = jnp.sum(pos * pos, axis=1, keepdims=True)
    ones = jnp.ones_like(x2)
    z = jnp.zeros_like(pos[:, 0:2])
    out_ref[...] = jnp.concatenate([pos, x2, sq, ones, z], axis=1)


def _k0b_body(pos_ref, x_ref, b_ref, dt_ref, out_ref):
    pos = pos_ref[...]
    sp = jnp.abs(x_ref[:, 2:3])
    b = b_ref[...]
    nd = jnp.zeros_like(sp)
    for bb in range(8):
        nd = jnp.where(b == bb, dt_ref[0:1, bb:bb + 1], nd)
    nd = jnp.maximum(nd, 0.01)
    z = jnp.zeros_like(pos)
    out_ref[...] = jnp.concatenate([pos, sp, nd, z], axis=1)


def _b1_body(a0_ref, a1_ref, pos_ref, x_ref, lo_ref, out_ref):
    i = pl.program_id(0)
    acc = a0_ref[...] + a1_ref[...]
    cnt = acc[:, 5:6]
    dc = jnp.maximum(cnt, 1.0)
    mp = acc[:, 0:3] / dc
    mi = acc[:, 3:4] / dc
    p = pos_ref[...]
    res_pos = jnp.sum((p - mp) ** 2, axis=1, keepdims=True)
    res_int = (x_ref[:, 2:3] - mi) ** 2
    lv = jnp.clip(lo_ref[...], _L_MIN, _L_MAX)
    prec = jnp.exp(-lv)
    tot = jnp.sum(0.5 * prec * res_pos + 0.5 * prec * res_int + 0.5 * lv)
    @pl.when(i == 0)
    def _():
        out_ref[...] = jnp.zeros((1, 1), jnp.float32)
    out_ref[...] += jnp.reshape(tot, (1, 1))


def _b2_body(gs_ref, gd_ref, gt_ref, out_ref):
    ps = gs_ref[:, 0:3]
    sp = gs_ref[:, 3:4]
    nd = gs_ref[:, 4:5]
    pd = gd_ref[:, 0:3]
    mv = pd - ps
    nrm = jnp.sqrt(jnp.sum(mv * mv, axis=1, keepdims=True))
    unit = mv / (nrm + 1e-9)
    pred = ps + sp * unit * nd
    a2 = jnp.sum(pred * pred, axis=1, keepdims=True)
    g = gt_ref[...]
    b2 = jnp.sum(g * g, axis=1)[None, :]
    d2 = jnp.maximum(
        a2 + b2 - 2.0 * jnp.dot(pred, g.T, preferred_element_type=jnp.float32),
        0.0)
    out_ref[...] = jnp.min(d2, axis=1, keepdims=True)


def _b3_body(a0_ref, a1_ref, pos_ref, ro_ref, tr_ref, den_ref, s_ref):
    i = pl.program_id(0)
    acc = a0_ref[...] + a1_ref[...]
    s2 = acc[:, 0:3]
    s1 = acc[:, 4:5]
    cnt = acc[:, 5:6]
    p = pos_ref[...]
    rsq = jnp.sum(p * p, axis=1, keepdims=True)
    sum_d = cnt * rsq - 2.0 * jnp.sum(p * s2, axis=1, keepdims=True) + s1
    val = sum_d / jnp.maximum(cnt, 1.0) ** 2
    spat = jnp.where(cnt > 0, val, _GHOST)
    lv = jnp.clip(ro_ref[...], _R_MIN, _R_MAX)
    ndt = tr_ref[:, 4:5]
    den = 2.0 * jnp.exp(lv) * ndt * ndt + 1e-9
    den_ref[...] = den
    tot = jnp.sum(spat / den + 0.5 * lv)
    @pl.when(i == 0)
    def _():
        s_ref[...] = jnp.zeros((1, 1), jnp.float32)
    s_ref[...] += jnp.reshape(tot, (1, 1))


def _d_body(sl_ref, s1_ref, s2_ref, ph_ref, out_ref):
    ph1 = jnp.sum(ph_ref[0:1, :], axis=1, keepdims=True)
    ph2 = jnp.sum(ph_ref[1:2, :], axis=1, keepdims=True)
    out_ref[...] = (sl_ref[...] / float(_NL)
                    + (s1_ref[...] + ph1) / float(_NR)
                    + (s2_ref[...] + ph2) / float(_NR))


def _blk(shape, imap):
    return pl.BlockSpec(shape, imap)


def kernel(lidar_out, lidar_pos, lidar_x, lidar_spatial_edge_index, radar1_out,
           radar1_pos, radar1_x, radar1_batch, radar1_temporal_edge_index,
           radar1_to_lidar_src, radar1_to_lidar_dst, radar2_out, radar2_pos,
           radar2_x, radar2_batch, radar2_temporal_edge_index,
           radar2_to_lidar_src, radar2_to_lidar_dst, dt_sec, gt_radar_pos):
    f32, i32 = jnp.float32, jnp.int32

    lid_ei = lidar_spatial_edge_index.astype(i32)
    pad_c = _ECP - 200000
    pad_t = _ETP - 100000
    cpad_s = _NR + (jnp.arange(pad_c, dtype=i32) % (_NRP - _NR))
    cpad_d = jnp.arange(pad_c, dtype=i32) % _NL
    r1cs = jnp.concatenate([radar1_to_lidar_src.astype(i32), cpad_s])
    r1cd = jnp.concatenate([radar1_to_lidar_dst.astype(i32), cpad_d])
    r2cs = jnp.concatenate([radar2_to_lidar_src.astype(i32), cpad_s])
    r2cd = jnp.concatenate([radar2_to_lidar_dst.astype(i32), cpad_d])
    tpad = jnp.broadcast_to(jnp.arange(pad_t, dtype=i32) % _NR, (2, pad_t))
    tei1 = jnp.concatenate([radar1_temporal_edge_index.astype(i32), tpad], axis=1)
    tei2 = jnp.concatenate([radar2_temporal_edge_index.astype(i32), tpad], axis=1)
    zeros = jnp.zeros((_NL, _D), f32)
    dt2 = dt_sec.reshape(1, 8).astype(f32)

    t_lid = pl.pallas_call(
        _k0a_body,
        grid=(100,),
        in_specs=[_blk((1000, 3), lambda i: (i, 0)),
                  _blk((1000, 3), lambda i: (i, 0))],
        out_specs=_blk((1000, _D), lambda i: (i, 0)),
        out_shape=jax.ShapeDtypeStruct((_NL, _D), f32),
    )(lidar_pos, lidar_x)

    rpos = jnp.concatenate([radar1_pos, radar2_pos], axis=0)
    rx = jnp.concatenate([radar1_x, radar2_x], axis=0)
    rb = jnp.concatenate([radar1_batch.astype(i32),
                          radar2_batch.astype(i32)]).reshape(2 * _NR, 1)
    t_rad = pl.pallas_call(
        _k0b_body,
        grid=(40,),
        in_specs=[_blk((1000, 3), lambda i: (i, 0)),
                  _blk((1000, 3), lambda i: (i, 0)),
                  _blk((1000, 1), lambda i: (i, 0)),
                  _blk((1, 8), lambda i: (0, 0))],
        out_specs=_blk((1000, _D), lambda i: (i, 0)),
        out_shape=jax.ShapeDtypeStruct((2 * _NR, _D), f32),
    )(rpos, rx, rb, dt2)
    t_r1 = t_rad[:_NR]
    t_r2 = t_rad[_NR:]

    lid_acc, r1_acc, r2_acc, g1s, g1d, g2s, g2d = _sc_edge_pass(
        t_lid, t_r1, t_r2, lid_ei, r1cs, r1cd, r2cs, r2cd, tei1, tei2, zeros)

    s_lid = pl.pallas_call(
        _b1_body,
        grid=(100,),
        in_specs=[_blk((1000, _D), lambda i: (i, 0)),
                  _blk((1000, _D), lambda i: (i, 0)),
                  _blk((1000, 3), lambda i: (i, 0)),
                  _blk((1000, 3), lambda i: (i, 0)),
                  _blk((1000, 1), lambda i: (i, 0))],
        out_specs=_blk((1, 1), lambda i: (0, 0)),
        out_shape=jax.ShapeDtypeStruct((1, 1), f32),
    )(lid_acc[0], lid_acc[1], lidar_pos, lidar_x, lidar_out)

    def run_b2(gs, gd):
        return pl.pallas_call(
            _b2_body,
            grid=(_ETP // 1000,),
            in_specs=[_blk((1000, _D), lambda i: (i, 0)),
                      _blk((1000, _D), lambda i: (i, 0)),
                      _blk((_NGT, 3), lambda i: (0, 0))],
            out_specs=_blk((1000, 1), lambda i: (i, 0)),
            out_shape=jax.ShapeDtypeStruct((_ETP, 1), f32),
        )(gs, gd, gt_radar_pos)
    md1 = run_b2(g1s, g1d)
    md2 = run_b2(g2s, g2d)

    def run_b3(racc, pos, rout, t_r):
        return pl.pallas_call(
            _b3_body,
            grid=(20,),
            in_specs=[_blk((1000, _D), lambda i: (i, 0)),
                      _blk((1000, _D), lambda i: (i, 0)),
                      _blk((1000, 3), lambda i: (i, 0)),
                      _blk((1000, 1), lambda i: (i, 0)),
                      _blk((1000, _D), lambda i: (i, 0))],
            out_specs=[_blk((1000, 1), lambda i: (i, 0)),
                       _blk((1, 1), lambda i: (0, 0))],
            out_shape=[jax.ShapeDtypeStruct((_NR, 1), f32),
                       jax.ShapeDtypeStruct((1, 1), f32)],
        )(racc[0, :_NR], racc[1, :_NR], pos, rout, t_r)
    den1, s_r1 = run_b3(r1_acc, radar1_pos, radar1_out, t_r1)
    den2, s_r2 = run_b3(r2_acc, radar2_pos, radar2_out, t_r2)

    phys = _sc_phys_pass(tei1, md1.reshape(_ETP), den1.reshape(_NR),
                         tei2, md2.reshape(_ETP), den2.reshape(_NR))

    tot = pl.pallas_call(
        _d_body,
        grid=(1,),
        in_specs=[_blk((1, 1), lambda i: (0, 0)),
                  _blk((1, 1), lambda i: (0, 0)),
                  _blk((1, 1), lambda i: (0, 0)),
                  _blk((2, 16), lambda i: (0, 0))],
        out_specs=_blk((1, 1), lambda i: (0, 0)),
        out_shape=jax.ShapeDtypeStruct((1, 1), f32),
    )(s_lid, s_r1, s_r2, phys)
    return tot[0, 0]
